# baseline (speedup 1.0000x reference)
import functools

import jax
import jax.numpy as jnp
from jax import lax
from jax.experimental import pallas as pl
from jax.experimental.pallas import tpu as pltpu
from jax.experimental.pallas import tpu_sc as plsc

L = 200
D = 64
B = 4096
HALF = 100
LANES = 16
VPR = D // LANES


def _make_kernel():
    info = plsc.get_sparse_core_info()
    nc, ns = info.num_cores, info.num_subcores
    nw = nc * ns
    rpw = B // nw

    mesh = plsc.VectorSubcoreMesh(core_axis_name="c", subcore_axis_name="s")

    @functools.partial(
        pl.kernel,
        mesh=mesh,
        out_type=jax.ShapeDtypeStruct((B, L, D), jnp.float32),
        scratch_types=[
            pltpu.VMEM((L, D), jnp.float32),
            pltpu.VMEM((2, HALF), jnp.int32),
            pltpu.VMEM((L, D), jnp.float32),
            pltpu.SemaphoreType.DMA,
        ],
        compiler_params=pltpu.CompilerParams(use_tc_tiling_on_sc=False),
    )
    def k(idx_hbm, tok_hbm, pos_hbm, out_hbm, pos_v, idx_v, rows_v, sem):
        wid = lax.axis_index("s") * nc + lax.axis_index("c")
        base = wid * rpw
        pltpu.sync_copy(pos_hbm, pos_v)

        def row_body(t, carry):
            b = base + t
            pltpu.sync_copy(idx_hbm.at[b], idx_v)
            cp0 = pltpu.async_copy(
                tok_hbm.at[idx_v.at[0]], rows_v.at[pl.ds(0, HALF)], sem)
            cp1 = pltpu.async_copy(
                tok_hbm.at[idx_v.at[1]], rows_v.at[pl.ds(HALF, HALF)], sem)
            cp0.wait()
            cp1.wait()

            def add_body(r, c2):
                for c in range(VPR):
                    s = pl.ds(c * LANES, LANES)
                    rows_v[r, s] = rows_v[r, s] + pos_v[r, s]
                return c2

            lax.fori_loop(0, L, add_body, 0)
            pltpu.sync_copy(rows_v, out_hbm.at[b])
            return carry

        lax.fori_loop(0, rpw, row_body, 0)

    return k


_kernel = _make_kernel()


@jax.jit
def kernel(inputs, token_table, position_table):
    idx = inputs.astype(jnp.int32).reshape(B, 2, HALF)
    return _kernel(idx, token_table, position_table)


# speedup vs baseline: 1.1509x; 1.1509x over previous
import functools

import jax
import jax.numpy as jnp
from jax import lax
from jax.experimental import pallas as pl
from jax.experimental.pallas import tpu as pltpu
from jax.experimental.pallas import tpu_sc as plsc

L = 200
D = 64
B = 4096
HALF = 100
LANES = 16
VPR = D // LANES
NBUF = 4


def _make_kernel():
    info = plsc.get_sparse_core_info()
    nc, ns = info.num_cores, info.num_subcores
    nw = nc * ns
    rpw = B // nw
    nchunk = rpw // NBUF

    mesh = plsc.VectorSubcoreMesh(core_axis_name="c", subcore_axis_name="s")

    @functools.partial(
        pl.kernel,
        mesh=mesh,
        out_type=jax.ShapeDtypeStruct((B, L, D), jnp.float32),
        scratch_types=[
            pltpu.VMEM((L, D), jnp.float32),
            pltpu.VMEM((rpw, 2, HALF), jnp.int32),
            pltpu.VMEM((NBUF, L, D), jnp.float32),
            pltpu.SemaphoreType.DMA,
            pltpu.SemaphoreType.DMA,
        ],
        compiler_params=pltpu.CompilerParams(use_tc_tiling_on_sc=False),
    )
    def k(idx_hbm, tok_hbm, pos_hbm, out_hbm, pos_v, idx_all, rows_v, gsem,
          ssem):
        wid = lax.axis_index("s") * nc + lax.axis_index("c")
        base = wid * rpw
        pltpu.sync_copy(pos_hbm, pos_v)
        pltpu.sync_copy(idx_hbm.at[pl.ds(base, rpw)], idx_all)

        def issue_gathers(t, b):
            for h in range(2):
                pltpu.async_copy(
                    tok_hbm.at[idx_all.at[t, h]],
                    rows_v.at[b, pl.ds(h * HALF, HALF)], gsem)

        def wait_gathers(t, b):
            for h in range(2):
                pltpu.make_async_copy(
                    tok_hbm.at[idx_all.at[t, h]],
                    rows_v.at[b, pl.ds(h * HALF, HALF)], gsem).wait()

        def issue_store(t, b):
            pltpu.async_copy(rows_v.at[b], out_hbm.at[base + t], ssem)

        def wait_store(t, b):
            pltpu.make_async_copy(
                rows_v.at[b], out_hbm.at[base + t], ssem).wait()

        for b in range(NBUF):
            issue_gathers(b, b)

        def chunk_body(g, carry):
            t0 = g * NBUF
            for b in range(NBUF):
                wait_gathers(t0 + b, b)

            def add_body(r, c2):
                for c in range(VPR):
                    s = pl.ds(c * LANES, LANES)
                    p = pos_v[r, s]
                    for b in range(NBUF):
                        rows_v[b, r, s] = rows_v[b, r, s] + p
                return c2

            lax.fori_loop(0, L, add_body, 0)
            for b in range(NBUF):
                issue_store(t0 + b, b)

            @pl.when(g + 1 < nchunk)
            def _():
                for b in range(NBUF):
                    wait_store(t0 + b, b)
                    issue_gathers(t0 + NBUF + b, b)

            return carry

        lax.fori_loop(0, nchunk, chunk_body, 0)
        for b in range(NBUF):
            wait_store((nchunk - 1) * NBUF + b, b)

    return k


_kernel = _make_kernel()


@jax.jit
def kernel(inputs, token_table, position_table):
    idx = inputs.astype(jnp.int32).reshape(B, 2, HALF)
    return _kernel(idx, token_table, position_table)


# speedup vs baseline: 1.1554x; 1.0039x over previous
import functools

import jax
import jax.numpy as jnp
from jax import lax
from jax.experimental import pallas as pl
from jax.experimental.pallas import tpu as pltpu
from jax.experimental.pallas import tpu_sc as plsc

L = 200
D = 64
B = 4096
SPLIT = (104, 96)
LANES = 16
VPR = D // LANES
NBUF = 4


def _make_kernel():
    info = plsc.get_sparse_core_info()
    nc, ns = info.num_cores, info.num_subcores
    nw = nc * ns
    rpw = B // nw
    nchunk = rpw // NBUF

    mesh = plsc.VectorSubcoreMesh(core_axis_name="c", subcore_axis_name="s")

    @functools.partial(
        pl.kernel,
        mesh=mesh,
        out_type=jax.ShapeDtypeStruct((B, L, D), jnp.float32),
        scratch_types=[
            pltpu.VMEM((L, D), jnp.float32),
            pltpu.VMEM((rpw, L), jnp.int32),
            pltpu.VMEM((NBUF, L, D), jnp.float32),
            pltpu.SemaphoreType.DMA,
            pltpu.SemaphoreType.DMA,
        ],
        compiler_params=pltpu.CompilerParams(use_tc_tiling_on_sc=False),
    )
    def k(idx_hbm, tok_hbm, pos_hbm, out_hbm, pos_v, idx_all, rows_v, gsem,
          ssem):
        wid = lax.axis_index("s") * nc + lax.axis_index("c")
        base = wid * rpw
        pltpu.sync_copy(pos_hbm, pos_v)
        pltpu.sync_copy(idx_hbm.at[pl.ds(base, rpw)], idx_all)

        def gather_pairs(t, b):
            off = 0
            pairs = []
            for n in SPLIT:
                pairs.append((
                    tok_hbm.at[idx_all.at[t, pl.ds(off, n)]],
                    rows_v.at[b, pl.ds(off, n)],
                ))
                off += n
            return pairs

        def issue_gathers(t, b):
            for src, dst in gather_pairs(t, b):
                pltpu.async_copy(src, dst, gsem)

        def wait_gathers(t, b):
            for src, dst in gather_pairs(t, b):
                pltpu.make_async_copy(src, dst, gsem).wait()

        def issue_store(t, b):
            pltpu.async_copy(rows_v.at[b], out_hbm.at[base + t], ssem)

        def wait_store(t, b):
            pltpu.make_async_copy(
                rows_v.at[b], out_hbm.at[base + t], ssem).wait()

        for b in range(NBUF):
            issue_gathers(b, b)

        def chunk_body(g, carry):
            t0 = g * NBUF
            for b in range(NBUF):
                wait_gathers(t0 + b, b)

            def add_body(r, c2):
                for c in range(VPR):
                    s = pl.ds(c * LANES, LANES)
                    p = pos_v[r, s]
                    for b in range(NBUF):
                        rows_v[b, r, s] = rows_v[b, r, s] + p
                return c2

            lax.fori_loop(0, L, add_body, 0)
            for b in range(NBUF):
                issue_store(t0 + b, b)

            @pl.when(g + 1 < nchunk)
            def _():
                for b in range(NBUF):
                    wait_store(t0 + b, b)
                    issue_gathers(t0 + NBUF + b, b)

            return carry

        lax.fori_loop(0, nchunk, chunk_body, 0)
        for b in range(NBUF):
            wait_store((nchunk - 1) * NBUF + b, b)

    return k


_kernel = _make_kernel()


@jax.jit
def kernel(inputs, token_table, position_table):
    return _kernel(inputs.astype(jnp.int32), token_table, position_table)
